# baseline (speedup 1.0000x reference)
import jax
import jax.numpy as jnp
from jax.experimental import pallas as pl
from jax.experimental.pallas import tpu as pltpu

_BV = 2560


def _mask_mul_kernel(step_ref, prob_ref, mask_ref, out_ref):
    col = jax.lax.rem(step_ref[0] - 1, 8)
    mt = jnp.transpose(mask_ref[...], (1, 0))
    sel = jax.lax.broadcasted_iota(jnp.int32, mt.shape, 1) == col
    m = jnp.sum(jnp.where(sel, mt, 0.0), axis=1, keepdims=True)
    out_ref[...] = prob_ref[...] * m


def kernel(probabilites, mask, step):
    B, V = probabilites.shape
    S = mask.shape[0]
    pt = probabilites.T
    step_arr = jnp.atleast_1d(jnp.asarray(step, jnp.int32))
    grid = ((V + _BV - 1) // _BV,)
    grid_spec = pltpu.PrefetchScalarGridSpec(
        num_scalar_prefetch=1,
        grid=grid,
        in_specs=[
            pl.BlockSpec((_BV, B), lambda i, s: (i, 0)),
            pl.BlockSpec((8, _BV), lambda i, s: ((s[0] - 1) // 8, i)),
        ],
        out_specs=pl.BlockSpec((_BV, B), lambda i, s: (i, 0)),
    )
    out_t = pl.pallas_call(
        _mask_mul_kernel,
        grid_spec=grid_spec,
        out_shape=jax.ShapeDtypeStruct((V, B), probabilites.dtype),
    )(step_arr, pt, mask)
    return out_t.T


# speedup vs baseline: 1.0036x; 1.0036x over previous
import jax
import jax.numpy as jnp
from jax.experimental import pallas as pl
from jax.experimental.pallas import tpu as pltpu

_BV = 3328


def _mask_mul_kernel(step_ref, prob_ref, mask_ref, out_ref):
    col = jax.lax.rem(step_ref[0] - 1, 8)
    mt = jnp.transpose(mask_ref[...], (1, 0))
    sel = jax.lax.broadcasted_iota(jnp.int32, mt.shape, 1) == col
    m = jnp.sum(jnp.where(sel, mt, 0.0), axis=1, keepdims=True)
    out_ref[...] = prob_ref[...] * m


def kernel(probabilites, mask, step):
    B, V = probabilites.shape
    S = mask.shape[0]
    pt = probabilites.T
    step_arr = jnp.atleast_1d(jnp.asarray(step, jnp.int32))
    grid = ((V + _BV - 1) // _BV,)
    grid_spec = pltpu.PrefetchScalarGridSpec(
        num_scalar_prefetch=1,
        grid=grid,
        in_specs=[
            pl.BlockSpec((_BV, B), lambda i, s: (i, 0)),
            pl.BlockSpec((8, _BV), lambda i, s: ((s[0] - 1) // 8, i)),
        ],
        out_specs=pl.BlockSpec((_BV, B), lambda i, s: (i, 0)),
    )
    out_t = pl.pallas_call(
        _mask_mul_kernel,
        grid_spec=grid_spec,
        out_shape=jax.ShapeDtypeStruct((V, B), probabilites.dtype),
    )(step_arr, pt, mask)
    return out_t.T


# speedup vs baseline: 1.0054x; 1.0019x over previous
import jax
import jax.numpy as jnp
from jax.experimental import pallas as pl
from jax.experimental.pallas import tpu as pltpu

_BV = 3072


def _mask_mul_kernel(step_ref, prob_ref, mask_ref, out_ref):
    col = jax.lax.rem(step_ref[0] - 1, 8)
    mt = jnp.transpose(mask_ref[...], (1, 0))
    sel = jax.lax.broadcasted_iota(jnp.int32, mt.shape, 1) == col
    m = jnp.sum(jnp.where(sel, mt, 0.0), axis=1, keepdims=True)
    out_ref[...] = prob_ref[...] * m


def kernel(probabilites, mask, step):
    B, V = probabilites.shape
    S = mask.shape[0]
    pt = probabilites.T
    step_arr = jnp.atleast_1d(jnp.asarray(step, jnp.int32))
    grid = ((V + _BV - 1) // _BV,)
    grid_spec = pltpu.PrefetchScalarGridSpec(
        num_scalar_prefetch=1,
        grid=grid,
        in_specs=[
            pl.BlockSpec((_BV, B), lambda i, s: (i, 0)),
            pl.BlockSpec((8, _BV), lambda i, s: ((s[0] - 1) // 8, i)),
        ],
        out_specs=pl.BlockSpec((_BV, B), lambda i, s: (i, 0)),
    )
    out_t = pl.pallas_call(
        _mask_mul_kernel,
        grid_spec=grid_spec,
        out_shape=jax.ShapeDtypeStruct((V, B), probabilites.dtype),
    )(step_arr, pt, mask)
    return out_t.T


# speedup vs baseline: 1.0055x; 1.0000x over previous
import jax
import jax.numpy as jnp
from jax.experimental import pallas as pl
from jax.experimental.pallas import tpu as pltpu

_BV = 3072


def _mask_mul_kernel(step_ref, prob_ref, mask_ref, out_ref):
    col = jax.lax.rem(step_ref[0] - 1, 8)
    mt = jnp.transpose(mask_ref[...], (1, 0))
    sel = jax.lax.broadcasted_iota(jnp.int32, mt.shape, 1) == col
    m = jnp.sum(jnp.where(sel, mt, 0.0), axis=1, keepdims=True)
    out_ref[...] = prob_ref[...] * m


def kernel(probabilites, mask, step):
    B, V = probabilites.shape
    pt = probabilites.T
    step_arr = jnp.atleast_1d(jnp.asarray(step, jnp.int32))
    grid = ((V + _BV - 1) // _BV,)
    grid_spec = pltpu.PrefetchScalarGridSpec(
        num_scalar_prefetch=1,
        grid=grid,
        in_specs=[
            pl.BlockSpec((_BV, B), lambda i, s: (i, 0)),
            pl.BlockSpec((8, _BV), lambda i, s: ((s[0] - 1) // 8, i)),
        ],
        out_specs=pl.BlockSpec((_BV, B), lambda i, s: (i, 0)),
    )
    out_t = pl.pallas_call(
        _mask_mul_kernel,
        grid_spec=grid_spec,
        out_shape=jax.ShapeDtypeStruct((V, B), probabilites.dtype),
    )(step_arr, pt, mask)
    return out_t.T
